# baseline (speedup 1.0000x reference)
import functools

import jax
import jax.numpy as jnp
from jax import lax
from jax.experimental import pallas as pl
from jax.experimental.pallas import tpu as pltpu
from jax.experimental.pallas import tpu_sc as plsc

N_CAND = 1000000
N_SAMP = 200000
NUM_SEG = 1024

NC, NS = 2, 16
NW = NC * NS
C = 31232
NCH = 8
CH = C // NCH
TAIL = N_CAND - NW * C
DEPTH = 3
S = 6272
NSP = NW * S
NBINS = 1040
NBND = 1056

_MESH = plsc.VectorSubcoreMesh(core_axis_name="c", subcore_axis_name="s")
_PARAMS = pltpu.CompilerParams(needs_layout_passes=False)


def _wid():
    return lax.axis_index("s") * NC + lax.axis_index("c")


def _seg_of(bnd_v, pos):
    def scan(i, cnt):
        w = bnd_v[pl.ds(i * 16, 16)]
        return cnt + jnp.sum(jnp.where(w <= pos, 1, 0))
    return lax.fori_loop(0, NBND // 16, scan, jnp.int32(0))


def _lane_segs(bnd_v, cur, p16):
    wa = bnd_v[pl.ds(cur, 16)]
    wb = bnd_v[pl.ds(cur + 16, 16)]
    cnt = jnp.zeros((16,), jnp.int32)
    for m in range(16):
        cnt = cnt + jnp.where(p16 >= wa[m], 1, 0) + jnp.where(p16 >= wb[m], 1, 0)
    return cur + cnt


def _pass1_body(eid_hbm, u_hbm, bnd_hbm, tab_hbm, e_hbm, pbins_hbm,
                e_v, eid_v, u_v, bnd_v, bins_v,
                sem_i, sem_u, sem_b, gs0, gs1, gs2, gs3, wsem):
    wid = _wid()
    base = wid * C
    gsems = [gs0, gs1, gs2, gs3]

    cp_i = pltpu.async_copy(eid_hbm.at[pl.ds(base, C)], eid_v, sem_i)
    cp_u = pltpu.async_copy(u_hbm.at[pl.ds(base, C)], u_v, sem_u)
    cp_b = pltpu.async_copy(bnd_hbm, bnd_v, sem_b)

    def zero_bins(i, _):
        bins_v[pl.ds(i * 16, 16)] = jnp.zeros((16,), jnp.float32)
        return _
    lax.fori_loop(0, NBINS // 16, zero_bins, None)

    lanes = lax.iota(jnp.int32, 16)
    lane0 = lanes == 0

    def gather(c, n=CH):
        sl = pl.ds(c * CH, n)
        return pltpu.async_copy(tab_hbm.at[eid_v.at[sl]], e_v.at[sl],
                                gsems[c % 4])

    cp_i.wait()
    g = {c: gather(c) for c in range(DEPTH)}
    cp_u.wait()
    cp_b.wait()

    def make_step(gbase, e_off):
        def step(j, carry):
            acc, cur, nxt = carry
            sl = pl.ds(e_off + j * 16, 16)
            e16 = jnp.exp(e_v[sl] + u_v[sl])
            e_v[sl] = e16
            p0 = gbase + j * 16

            def fast(_):
                return (acc + e16, cur, nxt)

            def slow(_):
                w = bins_v[pl.ds(cur, 16)]
                bins_v[pl.ds(cur, 16)] = w + jnp.where(lane0, jnp.sum(acc), 0.0)
                seg16 = _lane_segs(bnd_v, cur, p0 + lanes)
                plsc.addupdate_scatter(bins_v, [seg16], e16)
                cur2 = seg16[15]
                nxt2 = bnd_v[pl.ds(cur2, 16)][0]
                return (jnp.zeros_like(e16), cur2, nxt2)

            return lax.cond(p0 + 15 < nxt, fast, slow, 0)
        return step

    cur = _seg_of(bnd_v, base)
    nxt = bnd_v[pl.ds(cur, 16)][0]
    acc = jnp.zeros((16,), jnp.float32)
    wbs = []
    for c in range(NCH):
        if c + DEPTH < NCH:
            g[c + DEPTH] = gather(c + DEPTH)
        g[c].wait()
        acc, cur, nxt = lax.fori_loop(0, CH // 16,
                                      make_step(base + c * CH, c * CH),
                                      (acc, cur, nxt))
        wbs.append(pltpu.async_copy(e_v.at[pl.ds(c * CH, CH)],
                                    e_hbm.at[pl.ds(base + c * CH, CH)], wsem))

    w = bins_v[pl.ds(cur, 16)]
    bins_v[pl.ds(cur, 16)] = w + jnp.where(lane0, jnp.sum(acc), 0.0)

    for h in wbs:
        h.wait()

    @pl.when(wid == NW - 1)
    def _tail():
        tbase = NW * C
        ti = pltpu.async_copy(eid_hbm.at[pl.ds(tbase, TAIL)],
                              eid_v.at[pl.ds(0, TAIL)], sem_i)
        tu = pltpu.async_copy(u_hbm.at[pl.ds(tbase, TAIL)],
                              u_v.at[pl.ds(0, TAIL)], sem_u)
        ti.wait()
        pltpu.async_copy(tab_hbm.at[eid_v.at[pl.ds(0, TAIL)]],
                         e_v.at[pl.ds(0, TAIL)], gs0).wait()
        tu.wait()

        tcur = _seg_of(bnd_v, tbase)
        tnxt = bnd_v[pl.ds(tcur, 16)][0]
        tacc = jnp.zeros((16,), jnp.float32)
        tacc, tcur, tnxt = lax.fori_loop(0, TAIL // 16,
                                         make_step(tbase, 0),
                                         (tacc, tcur, tnxt))
        w = bins_v[pl.ds(tcur, 16)]
        bins_v[pl.ds(tcur, 16)] = w + jnp.where(lane0, jnp.sum(tacc), 0.0)
        pltpu.async_copy(e_v.at[pl.ds(0, TAIL)],
                         e_hbm.at[pl.ds(tbase, TAIL)], wsem).wait()

    pltpu.sync_copy(bins_v, pbins_hbm.at[wid])


def _pass2_body(e_hbm, bnd_hbm, pbins_hbm, ca_hbm, y_hbm,
                pb_v, bins_v, bnd_v, ca_v, e_v, y_v,
                sem_a, sem_b, sem_c, sem_d):
    wid = _wid()
    base = wid * S

    cp_ca = pltpu.async_copy(ca_hbm.at[pl.ds(base, S)], ca_v, sem_a)
    cp_pb = pltpu.async_copy(pbins_hbm, pb_v, sem_b)
    cp_bd = pltpu.async_copy(bnd_hbm, bnd_v, sem_d)
    cp_ca.wait()
    ge = pltpu.async_copy(e_hbm.at[ca_v], e_v, sem_c)
    cp_pb.wait()

    def red(i, _):
        sl = pl.ds(i * 16, 16)
        acc = pb_v[0, sl]

        def add_row(t, a):
            return a + pb_v[t, sl]
        bins_v[sl] = lax.fori_loop(1, NW, add_row, acc)
        return _
    lax.fori_loop(0, NBINS // 16, red, None)

    cp_bd.wait()
    ge.wait()

    cur = _seg_of(bnd_v, ca_v[pl.ds(0, 16)][0])
    nxt = bnd_v[pl.ds(cur, 16)][0]
    inv = (1.0 / bins_v[pl.ds(cur, 16)])[0]

    def step(j, carry):
        cur, nxt, inv = carry
        sl = pl.ds(j * 16, 16)
        c16 = ca_v[sl]
        e16 = e_v[sl]

        def fast(_):
            return (e16 * inv, cur, nxt, inv)

        def slow(_):
            seg16 = _lane_segs(bnd_v, cur, c16)
            denom = plsc.load_gather(bins_v, [seg16])
            cur2 = seg16[15]
            nxt2 = bnd_v[pl.ds(cur2, 16)][0]
            inv2 = (1.0 / bins_v[pl.ds(cur2, 16)])[0]
            return (e16 / denom, cur2, nxt2, inv2)

        y, cur, nxt, inv = lax.cond(c16[15] < nxt, fast, slow, 0)
        y_v[sl] = (1.0 - y) + y
        return (cur, nxt, inv)

    lax.fori_loop(0, S // 16, step, (cur, nxt, inv))
    pltpu.sync_copy(y_v, y_hbm.at[pl.ds(base, S)])


_pass1 = functools.partial(
    pl.kernel,
    out_type=(
        jax.ShapeDtypeStruct((N_CAND,), jnp.float32),
        jax.ShapeDtypeStruct((NW, NBINS), jnp.float32),
    ),
    mesh=_MESH,
    scratch_types=[
        pltpu.VMEM((C,), jnp.float32),
        pltpu.VMEM((C,), jnp.int32),
        pltpu.VMEM((C,), jnp.float32),
        pltpu.VMEM((NBND,), jnp.int32),
        pltpu.VMEM((NBINS,), jnp.float32),
        pltpu.SemaphoreType.DMA,
        pltpu.SemaphoreType.DMA,
        pltpu.SemaphoreType.DMA,
        pltpu.SemaphoreType.DMA,
        pltpu.SemaphoreType.DMA,
        pltpu.SemaphoreType.DMA,
        pltpu.SemaphoreType.DMA,
        pltpu.SemaphoreType.DMA,
    ],
    compiler_params=_PARAMS,
)(_pass1_body)

_pass2 = functools.partial(
    pl.kernel,
    out_type=jax.ShapeDtypeStruct((NSP,), jnp.float32),
    mesh=_MESH,
    scratch_types=[
        pltpu.VMEM((NW, NBINS), jnp.float32),
        pltpu.VMEM((NBINS,), jnp.float32),
        pltpu.VMEM((NBND,), jnp.int32),
        pltpu.VMEM((S,), jnp.int32),
        pltpu.VMEM((S,), jnp.float32),
        pltpu.VMEM((S,), jnp.float32),
        pltpu.SemaphoreType.DMA,
        pltpu.SemaphoreType.DMA,
        pltpu.SemaphoreType.DMA,
        pltpu.SemaphoreType.DMA,
    ],
    compiler_params=_PARAMS,
)(_pass2_body)


def kernel(candidate_edges, loglog_u, sampled_edges, edges_logits):
    eid = candidate_edges[:, 1]
    ca = sampled_edges[:, 5]
    bnd = jnp.searchsorted(candidate_edges[:, 0],
                           jnp.arange(1, NUM_SEG, dtype=jnp.int32),
                           method="scan_unrolled").astype(jnp.int32)
    bndp = jnp.concatenate([bnd, jnp.full((NBND - (NUM_SEG - 1),), N_CAND,
                                          jnp.int32)])
    cap = jnp.concatenate([ca, jnp.full((NSP - N_SAMP,), N_CAND - 1,
                                        jnp.int32)])

    e, pbins = _pass1(eid, loglog_u, bndp, edges_logits)
    ypad = _pass2(e, bndp, pbins, cap)
    return ypad[:N_SAMP]


# speedup vs baseline: 1.9084x; 1.9084x over previous
import functools

import jax
import jax.numpy as jnp
from jax import lax
from jax.experimental import pallas as pl
from jax.experimental.pallas import tpu as pltpu
from jax.experimental.pallas import tpu_sc as plsc

N_CAND = 1000000
N_SAMP = 200000
NUM_SEG = 1024

NC, NS = 2, 16
NW = NC * NS
C = 31360
NP = NW * C
NCH = 8
CH = C // NCH
DEPTH = 3
S = 6272
NSP = NW * S
NBINS = 1040

_MESH = plsc.VectorSubcoreMesh(core_axis_name="c", subcore_axis_name="s")
_PARAMS = pltpu.CompilerParams(needs_layout_passes=False)


def _wid():
    return lax.axis_index("s") * NC + lax.axis_index("c")


def _pass1_body(eid_hbm, u_hbm, eg_hbm, tab_hbm, e_hbm, pbins_hbm,
                eid_v, u_v, eg_v, e_v, bins_v,
                sem_a, sem_b, sem_c, gs0, gs1, gs2, gs3, wsem):
    wid = _wid()
    base = wid * C

    cp_eid = pltpu.async_copy(eid_hbm.at[pl.ds(base, C)], eid_v, sem_a)
    cp_u = pltpu.async_copy(u_hbm.at[pl.ds(base, C)], u_v, sem_b)
    cp_eg = pltpu.async_copy(eg_hbm.at[pl.ds(base, C)], eg_v, sem_c)

    def zero_bins(i, _):
        bins_v[pl.ds(i * 16, 16)] = jnp.zeros((16,), jnp.float32)
        return _
    lax.fori_loop(0, NBINS // 16, zero_bins, None)

    gsems = [gs0, gs1, gs2, gs3]
    cp_eid.wait()
    g = {}
    for j in range(DEPTH):
        sl = pl.ds(j * CH, CH)
        g[j] = pltpu.async_copy(tab_hbm.at[eid_v.at[sl]], e_v.at[sl],
                                gsems[j % 4])
    cp_u.wait()
    cp_eg.wait()

    lane0 = lax.iota(jnp.int32, 16) == 0
    acc = jnp.zeros((16,), jnp.float32)
    prev = eg_v[pl.ds(0, 16)][0]
    wbs = []
    for c in range(NCH):
        if c + DEPTH < NCH:
            sl = pl.ds((c + DEPTH) * CH, CH)
            g[c + DEPTH] = pltpu.async_copy(tab_hbm.at[eid_v.at[sl]],
                                            e_v.at[sl],
                                            gsems[(c + DEPTH) % 4])
        g[c].wait()

        def step(j, carry, c=c):
            acc, prev = carry
            sl = pl.ds(c * CH + j * 16, 16)
            e16 = jnp.exp(e_v[sl] + u_v[sl])
            e_v[sl] = e16
            seg16 = eg_v[sl]
            s0 = seg16[0]
            s15 = seg16[15]
            uniform = jnp.logical_and(s0 == s15, s0 == prev)
            boundary = s0 != s15

            @pl.when(jnp.logical_not(uniform))
            def _flush():
                w = bins_v[pl.ds(prev, 16)]
                bins_v[pl.ds(prev, 16)] = w + jnp.where(lane0, jnp.sum(acc), 0.0)

            @pl.when(boundary)
            def _scatter():
                plsc.addupdate_scatter(bins_v, [seg16], e16)

            acc_n = jnp.where(uniform, acc + e16,
                              jnp.where(boundary, jnp.zeros_like(e16), e16))
            prev_n = jnp.where(uniform, prev, s15)
            return (acc_n, prev_n)

        acc, prev = lax.fori_loop(0, CH // 16, step, (acc, prev))
        csl = pl.ds(c * CH, CH)
        wbs.append(pltpu.async_copy(e_v.at[csl],
                                    e_hbm.at[pl.ds(base + c * CH, CH)], wsem))

    w = bins_v[pl.ds(prev, 16)]
    bins_v[pl.ds(prev, 16)] = w + jnp.where(lane0, jnp.sum(acc), 0.0)
    pltpu.sync_copy(bins_v, pbins_hbm.at[wid])
    for h in wbs:
        h.wait()


def _pass2_body(e_hbm, eg_hbm, pbins_hbm, ca_hbm, y_hbm,
                pb_v, bins_v, ca_v, e_v, seg_v, y_v, sem_a, sem_b, sem_c, sem_d):
    wid = _wid()
    base = wid * S

    cp_ca = pltpu.async_copy(ca_hbm.at[pl.ds(base, S)], ca_v, sem_a)
    cp_pb = pltpu.async_copy(pbins_hbm, pb_v, sem_b)
    cp_ca.wait()
    ge = pltpu.async_copy(e_hbm.at[ca_v], e_v, sem_c)
    gs = pltpu.async_copy(eg_hbm.at[ca_v], seg_v, sem_d)
    cp_pb.wait()

    def red(i, _):
        sl = pl.ds(i * 16, 16)
        acc = pb_v[0, sl]

        def add_row(t, a):
            return a + pb_v[t, sl]
        bins_v[sl] = lax.fori_loop(1, NW, add_row, acc)
        return _
    lax.fori_loop(0, NBINS // 16, red, None)

    ge.wait()
    gs.wait()

    def step(j, _):
        b = j * 16
        denom = plsc.load_gather(bins_v, [seg_v[pl.ds(b, 16)]])
        y = e_v[pl.ds(b, 16)] / denom
        y_v[pl.ds(b, 16)] = (1.0 - y) + y
        return _
    lax.fori_loop(0, S // 16, step, None)

    pltpu.sync_copy(y_v, y_hbm.at[pl.ds(base, S)])


_pass1 = functools.partial(
    pl.kernel,
    out_type=(
        jax.ShapeDtypeStruct((NP,), jnp.float32),
        jax.ShapeDtypeStruct((NW, NBINS), jnp.float32),
    ),
    mesh=_MESH,
    scratch_types=[
        pltpu.VMEM((C,), jnp.int32),
        pltpu.VMEM((C,), jnp.float32),
        pltpu.VMEM((C,), jnp.int32),
        pltpu.VMEM((C,), jnp.float32),
        pltpu.VMEM((NBINS,), jnp.float32),
        pltpu.SemaphoreType.DMA,
        pltpu.SemaphoreType.DMA,
        pltpu.SemaphoreType.DMA,
        pltpu.SemaphoreType.DMA,
        pltpu.SemaphoreType.DMA,
        pltpu.SemaphoreType.DMA,
        pltpu.SemaphoreType.DMA,
        pltpu.SemaphoreType.DMA,
    ],
    compiler_params=_PARAMS,
)(_pass1_body)

_pass2 = functools.partial(
    pl.kernel,
    out_type=jax.ShapeDtypeStruct((NSP,), jnp.float32),
    mesh=_MESH,
    scratch_types=[
        pltpu.VMEM((NW, NBINS), jnp.float32),
        pltpu.VMEM((NBINS,), jnp.float32),
        pltpu.VMEM((S,), jnp.int32),
        pltpu.VMEM((S,), jnp.float32),
        pltpu.VMEM((S,), jnp.int32),
        pltpu.VMEM((S,), jnp.float32),
        pltpu.SemaphoreType.DMA,
        pltpu.SemaphoreType.DMA,
        pltpu.SemaphoreType.DMA,
        pltpu.SemaphoreType.DMA,
    ],
    compiler_params=_PARAMS,
)(_pass2_body)


def kernel(candidate_edges, loglog_u, sampled_edges, edges_logits):
    eg = candidate_edges[:, 0]
    eid = candidate_edges[:, 1]
    ca = sampled_edges[:, 5]

    pad = NP - N_CAND
    egp = jnp.concatenate([eg, jnp.full((pad,), NUM_SEG, jnp.int32)])
    eidp = jnp.concatenate([eid, jnp.zeros((pad,), jnp.int32)])
    up = jnp.concatenate([loglog_u, jnp.zeros((pad,), jnp.float32)])
    cap = jnp.concatenate([ca, jnp.zeros((NSP - N_SAMP,), jnp.int32)])

    e, pbins = _pass1(eidp, up, egp, edges_logits)
    ypad = _pass2(e, egp, pbins, cap)
    return ypad[:N_SAMP]


# speedup vs baseline: 1.9414x; 1.0173x over previous
import functools

import jax
import jax.numpy as jnp
from jax import lax
from jax.experimental import pallas as pl
from jax.experimental.pallas import tpu as pltpu
from jax.experimental.pallas import tpu_sc as plsc

N_CAND = 1000000
N_SAMP = 200000
NUM_SEG = 1024

NC, NS = 2, 16
NW = NC * NS
C = 31360
NP = NW * C
NCH = 8
CH = C // NCH
DEPTH = 2
S = 6272
NSP = NW * S
NBINS = 1040

_MESH = plsc.VectorSubcoreMesh(core_axis_name="c", subcore_axis_name="s")
_PARAMS = pltpu.CompilerParams(needs_layout_passes=False)


def _wid():
    return lax.axis_index("s") * NC + lax.axis_index("c")


def _pass1_body(eid_hbm, u_hbm, eg_hbm, tab_hbm, e_hbm, pbins_hbm,
                eid_v, u_v, eg_v, e_v, bins_v,
                sem_a, sem_b, sem_c, gs0, gs1, wsem):
    wid = _wid()
    base = wid * C

    cp_eid = pltpu.async_copy(eid_hbm.at[pl.ds(base, C)], eid_v, sem_a)
    cp_u = pltpu.async_copy(u_hbm.at[pl.ds(base, C)], u_v, sem_b)
    cp_eg = pltpu.async_copy(eg_hbm.at[pl.ds(base, C)], eg_v, sem_c)

    def zero_bins(i, _):
        bins_v[pl.ds(i * 16, 16)] = jnp.zeros((16,), jnp.float32)
        return _
    lax.fori_loop(0, NBINS // 16, zero_bins, None)

    gsems = [gs0, gs1]
    cp_eid.wait()
    g = {}
    for j in range(DEPTH):
        sl = pl.ds(j * CH, CH)
        g[j] = pltpu.async_copy(tab_hbm.at[eid_v.at[sl]], e_v.at[sl],
                                gsems[j % DEPTH])
    cp_u.wait()
    cp_eg.wait()

    lane0 = lax.iota(jnp.int32, 16) == 0
    acc = jnp.zeros((16,), jnp.float32)
    prev = eg_v[pl.ds(0, 16)][0]
    wbs = []
    for c in range(NCH):
        g[c].wait()
        if c + DEPTH < NCH:
            sl = pl.ds((c + DEPTH) * CH, CH)
            g[c + DEPTH] = pltpu.async_copy(tab_hbm.at[eid_v.at[sl]],
                                            e_v.at[sl],
                                            gsems[(c + DEPTH) % DEPTH])

        def step(j, carry, c=c):
            acc, prev = carry
            sl = pl.ds(c * CH + j * 16, 16)
            e16 = jnp.exp(e_v[sl] + u_v[sl])
            e_v[sl] = e16
            seg16 = eg_v[sl]
            s0 = seg16[0]
            s15 = seg16[15]
            uniform = jnp.logical_and(s0 == s15, s0 == prev)
            boundary = s0 != s15

            @pl.when(jnp.logical_not(uniform))
            def _flush():
                w = bins_v[pl.ds(prev, 16)]
                bins_v[pl.ds(prev, 16)] = w + jnp.where(lane0, jnp.sum(acc), 0.0)

            @pl.when(boundary)
            def _scatter():
                plsc.addupdate_scatter(bins_v, [seg16], e16)

            acc_n = jnp.where(uniform, acc + e16,
                              jnp.where(boundary, jnp.zeros_like(e16), e16))
            prev_n = jnp.where(uniform, prev, s15)
            return (acc_n, prev_n)

        acc, prev = lax.fori_loop(0, CH // 16, step, (acc, prev))
        csl = pl.ds(c * CH, CH)
        wbs.append(pltpu.async_copy(e_v.at[csl],
                                    e_hbm.at[pl.ds(base + c * CH, CH)], wsem))

    w = bins_v[pl.ds(prev, 16)]
    bins_v[pl.ds(prev, 16)] = w + jnp.where(lane0, jnp.sum(acc), 0.0)
    pltpu.sync_copy(bins_v, pbins_hbm.at[wid])
    for h in wbs:
        h.wait()


def _pass2_body(e_hbm, eg_hbm, pbins_hbm, ca_hbm, y_hbm,
                pb_v, bins_v, ca_v, e_v, seg_v, y_v, sem_a, sem_b, sem_c, sem_d):
    wid = _wid()
    base = wid * S

    cp_ca = pltpu.async_copy(ca_hbm.at[pl.ds(base, S)], ca_v, sem_a)
    cp_pb = pltpu.async_copy(pbins_hbm, pb_v, sem_b)
    cp_ca.wait()
    ge = pltpu.async_copy(e_hbm.at[ca_v], e_v, sem_c)
    gs = pltpu.async_copy(eg_hbm.at[ca_v], seg_v, sem_d)
    cp_pb.wait()

    def red(i, _):
        sl = pl.ds(i * 16, 16)
        acc = pb_v[0, sl]

        def add_row(t, a):
            return a + pb_v[t, sl]
        bins_v[sl] = lax.fori_loop(1, NW, add_row, acc)
        return _
    lax.fori_loop(0, NBINS // 16, red, None)

    ge.wait()
    gs.wait()

    def step(j, _):
        b = j * 16
        denom = plsc.load_gather(bins_v, [seg_v[pl.ds(b, 16)]])
        y = e_v[pl.ds(b, 16)] / denom
        y_v[pl.ds(b, 16)] = (1.0 - y) + y
        return _
    lax.fori_loop(0, S // 16, step, None)

    pltpu.sync_copy(y_v, y_hbm.at[pl.ds(base, S)])


_pass1 = functools.partial(
    pl.kernel,
    out_type=(
        jax.ShapeDtypeStruct((NP,), jnp.float32),
        jax.ShapeDtypeStruct((NW, NBINS), jnp.float32),
    ),
    mesh=_MESH,
    scratch_types=[
        pltpu.VMEM((C,), jnp.int32),
        pltpu.VMEM((C,), jnp.float32),
        pltpu.VMEM((C,), jnp.int32),
        pltpu.VMEM((C,), jnp.float32),
        pltpu.VMEM((NBINS,), jnp.float32),
        pltpu.SemaphoreType.DMA,
        pltpu.SemaphoreType.DMA,
        pltpu.SemaphoreType.DMA,
        pltpu.SemaphoreType.DMA,
        pltpu.SemaphoreType.DMA,
        pltpu.SemaphoreType.DMA,
    ],
    compiler_params=_PARAMS,
)(_pass1_body)

_pass2 = functools.partial(
    pl.kernel,
    out_type=jax.ShapeDtypeStruct((NSP,), jnp.float32),
    mesh=_MESH,
    scratch_types=[
        pltpu.VMEM((NW, NBINS), jnp.float32),
        pltpu.VMEM((NBINS,), jnp.float32),
        pltpu.VMEM((S,), jnp.int32),
        pltpu.VMEM((S,), jnp.float32),
        pltpu.VMEM((S,), jnp.int32),
        pltpu.VMEM((S,), jnp.float32),
        pltpu.SemaphoreType.DMA,
        pltpu.SemaphoreType.DMA,
        pltpu.SemaphoreType.DMA,
        pltpu.SemaphoreType.DMA,
    ],
    compiler_params=_PARAMS,
)(_pass2_body)


def kernel(candidate_edges, loglog_u, sampled_edges, edges_logits):
    eg = candidate_edges[:, 0]
    eid = candidate_edges[:, 1]
    ca = sampled_edges[:, 5]

    pad = NP - N_CAND
    egp = jnp.concatenate([eg, jnp.full((pad,), NUM_SEG, jnp.int32)])
    eidp = jnp.concatenate([eid, jnp.zeros((pad,), jnp.int32)])
    up = jnp.concatenate([loglog_u, jnp.zeros((pad,), jnp.float32)])
    cap = jnp.concatenate([ca, jnp.zeros((NSP - N_SAMP,), jnp.int32)])

    e, pbins = _pass1(eidp, up, egp, edges_logits)
    ypad = _pass2(e, egp, pbins, cap)
    return ypad[:N_SAMP]
